# baseline (speedup 1.0000x reference)
import functools

import jax
import jax.numpy as jnp
from jax import lax
from jax.experimental import pallas as pl
from jax.experimental.pallas import tpu as pltpu
from jax.experimental.pallas import tpu_sc as plsc

_NC = 2
_NS = 16
_NW = _NC * _NS

_PAIR = 2

_RING = 4


@functools.lru_cache(maxsize=None)
def _build_gather(batch: int, hist: int, vocab: int, d: int):
    chunk = _PAIR * hist
    per_w = batch // (_NW * _PAIR)
    n_groups = per_w // _RING
    half = n_groups // 2
    assert per_w == _RING * n_groups and n_groups == 2 * half

    mesh = plsc.VectorSubcoreMesh(core_axis_name="c", subcore_axis_name="s")
    n_sem = 2 * _RING

    @functools.partial(
        pl.kernel,
        mesh=mesh,
        out_type=jax.ShapeDtypeStruct((batch, hist, d), jnp.float32),
        scratch_types=[
            pltpu.VMEM((per_w, chunk), jnp.int32),
            pltpu.VMEM((n_sem, chunk, d), jnp.float32),
        ]
        + [pltpu.SemaphoreType.DMA] * (2 * n_sem),
    )
    def gather_kernel(table_hbm, idx_hbm, out_hbm, idx_v, rows_v, *sems):
        gsems = sems[:n_sem]
        wsems = sems[n_sem:]
        wid = lax.axis_index("s") * _NC + lax.axis_index("c")
        base = wid * per_w
        pltpu.sync_copy(idx_hbm.at[wid], idx_v)

        def g_src(i):
            return table_hbm.at[idx_v.at[i]]

        def wb_start(i, s):
            for e in range(_PAIR):
                pltpu.async_copy(
                    rows_v.at[s, pl.ds(e * hist, hist)],
                    out_hbm.at[(base + i) * _PAIR + e],
                    wsems[s],
                )

        def wb_wait(i, s):
            for e in range(_PAIR):
                pltpu.make_async_copy(
                    rows_v.at[s, pl.ds(e * hist, hist)],
                    out_hbm.at[(base + i) * _PAIR + e],
                    wsems[s],
                ).wait()

        for b in range(_RING):
            pltpu.async_copy(g_src(b), rows_v.at[b], gsems[b])

        def outer(gg, carry):
            for p in range(2):
                g = 2 * gg + p
                for b in range(_RING):
                    s = p * _RING + b
                    sn = (1 - p) * _RING + b
                    i = g * _RING + b
                    pltpu.make_async_copy(g_src(i), rows_v.at[s], gsems[s]).wait()
                    wb_start(i, s)

                    def prefetch(i=i, sn=sn):
                        wb_wait(i - _RING, sn)
                        pltpu.async_copy(g_src(i + _RING), rows_v.at[sn], gsems[sn])

                    if p == 0:
                        @pl.when(gg > 0)
                        def _(i=i, sn=sn):
                            wb_wait(i - _RING, sn)

                        pltpu.async_copy(g_src(i + _RING), rows_v.at[sn], gsems[sn])
                    else:
                        pl.when(gg < half - 1)(prefetch)
            return carry

        lax.fori_loop(0, half, outer, 0)

        for b in range(_RING):
            wb_wait((n_groups - 2) * _RING + b, b)
            wb_wait((n_groups - 1) * _RING + b, _RING + b)

    return gather_kernel


_SPLIT = 2


@jax.jit
def kernel(x, emb_matrix):
    b, h = x.shape
    vocab, d = emb_matrix.shape
    bs = b // _SPLIT
    idx = x.reshape(_SPLIT, _NW, bs // (_NW * _PAIR), _PAIR * h)
    gather = _build_gather(bs, h, vocab, d)
    return jnp.concatenate(
        [gather(emb_matrix, idx[s]) for s in range(_SPLIT)], axis=0
    )


# speedup vs baseline: 1.6075x; 1.6075x over previous
import functools

import jax
import jax.numpy as jnp
from jax import lax
from jax.experimental import pallas as pl
from jax.experimental.pallas import tpu as pltpu
from jax.experimental.pallas import tpu_sc as plsc

_NC = 2
_NS = 16
_NW = _NC * _NS

_PAIR = 2

_RING = 4


@functools.lru_cache(maxsize=None)
def _build_gather(batch: int, hist: int, vocab: int, d: int):
    chunk = _PAIR * hist
    per_w = batch // (_NW * _PAIR)
    n_groups = per_w // _RING
    half = n_groups // 2
    assert per_w == _RING * n_groups and n_groups == 2 * half

    mesh = plsc.VectorSubcoreMesh(core_axis_name="c", subcore_axis_name="s")
    n_sem = 2 * _RING

    @functools.partial(
        pl.kernel,
        mesh=mesh,
        out_type=jax.ShapeDtypeStruct((batch, hist, d), jnp.float32),
        scratch_types=[
            pltpu.VMEM((per_w, chunk), jnp.int32),
            pltpu.VMEM((n_sem, chunk, d), jnp.float32),
        ]
        + [pltpu.SemaphoreType.DMA] * (2 * n_sem),
    )
    def gather_kernel(table_hbm, idx_hbm, out_hbm, idx_v, rows_v, *sems):
        gsems = sems[:n_sem]
        wsems = sems[n_sem:]
        wid = lax.axis_index("s") * _NC + lax.axis_index("c")
        base = wid * per_w
        pltpu.sync_copy(idx_hbm.at[wid], idx_v)

        def g_src(i):
            return table_hbm.at[idx_v.at[i]]

        def wb_start(i, s):
            for e in range(_PAIR):
                pltpu.async_copy(
                    rows_v.at[s, pl.ds(e * hist, hist)],
                    out_hbm.at[(base + i) * _PAIR + e],
                    wsems[s],
                )

        def wb_wait(i, s):
            for e in range(_PAIR):
                pltpu.make_async_copy(
                    rows_v.at[s, pl.ds(e * hist, hist)],
                    out_hbm.at[(base + i) * _PAIR + e],
                    wsems[s],
                ).wait()

        for b in range(_RING):
            pltpu.async_copy(g_src(b), rows_v.at[b], gsems[b])

        def outer(gg, carry):
            for p in range(2):
                g = 2 * gg + p
                for b in range(_RING):
                    s = p * _RING + b
                    sn = (1 - p) * _RING + b
                    i = g * _RING + b
                    pltpu.make_async_copy(g_src(i), rows_v.at[s], gsems[s]).wait()
                    wb_start(i, s)

                    def prefetch(i=i, sn=sn):
                        wb_wait(i - _RING, sn)
                        pltpu.async_copy(g_src(i + _RING), rows_v.at[sn], gsems[sn])

                    if p == 0:
                        @pl.when(gg > 0)
                        def _(i=i, sn=sn):
                            wb_wait(i - _RING, sn)

                        pltpu.async_copy(g_src(i + _RING), rows_v.at[sn], gsems[sn])
                    else:
                        pl.when(gg < half - 1)(prefetch)
            return carry

        lax.fori_loop(0, half, outer, 0)

        for b in range(_RING):
            wb_wait((n_groups - 2) * _RING + b, b)
            wb_wait((n_groups - 1) * _RING + b, _RING + b)

    return gather_kernel


@jax.jit
def kernel(x, emb_matrix):
    b, h = x.shape
    vocab, d = emb_matrix.shape
    idx = x.reshape(_NW, b // (_NW * _PAIR), _PAIR * h)
    return _build_gather(b, h, vocab, d)(emb_matrix, idx)
